# baseline (speedup 1.0000x reference)
import functools

import jax
import jax.numpy as jnp
from jax import lax
from jax.experimental import pallas as pl
from jax.experimental.pallas import tpu as pltpu
from jax.experimental.pallas import tpu_sc as plsc

D_MODEL_DIM = 64
SCALE = 8.0

_info = plsc.get_sparse_core_info()
NUM_CORES = _info.num_cores
NUM_SUBCORES = _info.num_subcores
NUM_LANES = _info.num_lanes
NUM_WORKERS = NUM_CORES * NUM_SUBCORES

CHUNK = 1024


def _make_kernel(total_rows: int):
    assert total_rows % (NUM_WORKERS * CHUNK) == 0
    per_worker = total_rows // NUM_WORKERS
    n_chunks = per_worker // CHUNK
    mesh = plsc.VectorSubcoreMesh(core_axis_name="c", subcore_axis_name="s")

    @functools.partial(
        pl.kernel,
        mesh=mesh,
        compiler_params=pltpu.CompilerParams(use_tc_tiling_on_sc=False),
        out_type=jax.ShapeDtypeStruct((total_rows, D_MODEL_DIM), jnp.float32),
        scratch_types=[
            pltpu.VMEM((CHUNK,), jnp.int32),
            pltpu.VMEM((CHUNK, D_MODEL_DIM), jnp.float32),
            pltpu.SemaphoreType.DMA,
        ],
    )
    def k(idx_hbm, table_hbm, out_hbm, idx_v, rows_v, sem):
        wid = lax.axis_index("s") * NUM_CORES + lax.axis_index("c")
        base = wid * per_worker

        def chunk_body(g, _):
            off = base + g * CHUNK
            pltpu.sync_copy(idx_hbm.at[pl.ds(off, CHUNK)], idx_v)
            pltpu.async_copy(table_hbm.at[idx_v], rows_v, sem).wait()

            def scale_row(i, _):
                for j in range(D_MODEL_DIM // NUM_LANES):
                    sl = pl.ds(j * NUM_LANES, NUM_LANES)
                    rows_v[i, sl] = rows_v[i, sl] * SCALE
                return 0

            lax.fori_loop(0, CHUNK, scale_row, 0)
            pltpu.sync_copy(rows_v, out_hbm.at[pl.ds(off, CHUNK)])
            return 0

        lax.fori_loop(0, n_chunks, chunk_body, 0)

    return k


def kernel(x, table):
    orig_shape = x.shape
    idx = x.reshape(-1).astype(jnp.int32)
    total_rows = idx.shape[0]
    out = _make_kernel(total_rows)(idx, table)
    return out.reshape(*orig_shape, D_MODEL_DIM)


# speedup vs baseline: 1.1068x; 1.1068x over previous
import functools

import jax
import jax.numpy as jnp
from jax import lax
from jax.experimental import pallas as pl
from jax.experimental.pallas import tpu as pltpu
from jax.experimental.pallas import tpu_sc as plsc

D_MODEL_DIM = 64
SCALE = 8.0

_info = plsc.get_sparse_core_info()
NUM_CORES = _info.num_cores
NUM_SUBCORES = _info.num_subcores
NUM_LANES = _info.num_lanes
NUM_WORKERS = NUM_CORES * NUM_SUBCORES

CHUNK = 800
NBUF = 2


def _make_kernel(total_rows: int):
    assert total_rows % (NUM_WORKERS * CHUNK) == 0
    n_chunks = total_rows // (NUM_WORKERS * CHUNK)
    assert n_chunks % NBUF == 0
    mesh = plsc.VectorSubcoreMesh(core_axis_name="c", subcore_axis_name="s")

    @functools.partial(
        pl.kernel,
        mesh=mesh,
        compiler_params=pltpu.CompilerParams(use_tc_tiling_on_sc=False),
        out_type=jax.ShapeDtypeStruct((total_rows, D_MODEL_DIM), jnp.float32),
        scratch_types=[
            pltpu.VMEM((n_chunks, CHUNK), jnp.int32),
            pltpu.VMEM((CHUNK, D_MODEL_DIM), jnp.float32),
            pltpu.VMEM((CHUNK, D_MODEL_DIM), jnp.float32),
            pltpu.SemaphoreType.DMA,
            pltpu.SemaphoreType.DMA,
            pltpu.SemaphoreType.DMA,
            pltpu.SemaphoreType.DMA,
        ],
    )
    def k(idx_hbm, table_hbm, out_hbm, idx_v, rows0, rows1, sg0, sg1, so0, so1):
        wid = lax.axis_index("s") * NUM_CORES + lax.axis_index("c")
        rows = (rows0, rows1)
        sg = (sg0, sg1)
        so = (so0, so1)

        pltpu.sync_copy(idx_hbm.at[pl.ds(wid * n_chunks, n_chunks)], idx_v)

        def start_gather(g, b):
            pltpu.async_copy(table_hbm.at[idx_v.at[g]], rows[b], sg[b])

        def wait_gather(b):
            pltpu.make_async_copy(table_hbm.at[idx_v.at[0]], rows[b], sg[b]).wait()

        def start_out(g, b):
            off = (wid * n_chunks + g) * CHUNK
            pltpu.async_copy(rows[b], out_hbm.at[pl.ds(off, CHUNK)], so[b])

        def wait_out(b):
            off = wid * n_chunks * CHUNK
            pltpu.make_async_copy(rows[b], out_hbm.at[pl.ds(off, CHUNK)], so[b]).wait()

        def scale(b):
            r = rows[b]

            @plsc.parallel_loop(0, CHUNK, unroll=8)
            def _(i):
                for j in range(D_MODEL_DIM // NUM_LANES):
                    sl = pl.ds(j * NUM_LANES, NUM_LANES)
                    r[i, sl] = r[i, sl] * SCALE

        start_gather(0, 0)

        def pair_body(p, _):
            for b in range(NBUF):
                g = NBUF * p + b
                nb = 1 - b

                @pl.when(g + 1 < n_chunks)
                def _():
                    @pl.when(g >= 1)
                    def _():
                        wait_out(nb)

                    start_gather(g + 1, nb)

                wait_gather(b)
                scale(b)
                start_out(g, b)
            return 0

        lax.fori_loop(0, n_chunks // NBUF, pair_body, 0)
        wait_out(0)
        wait_out(1)

    return k


def kernel(x, table):
    orig_shape = x.shape
    idx = x.reshape(-1).astype(jnp.int32)
    total_rows = idx.shape[0]
    per_worker = total_rows // NUM_WORKERS
    idx2d = idx.reshape(total_rows // CHUNK, CHUNK)
    del per_worker
    out = _make_kernel(total_rows)(idx2d, table)
    return out.reshape(*orig_shape, D_MODEL_DIM)
